# baseline (speedup 1.0000x reference)
import jax
import jax.numpy as jnp
from jax.experimental import pallas as pl

_D = 256
_KX = _D // 2 + 1
_HALF = _D // 2


def _accum_vol(imgs, ctf, rotMats, hwShiftAngs):
    D_ = imgs.shape[-1]
    KX_ = D_ // 2 + 1
    half = D_ // 2
    f2d = jnp.fft.rfft2(jnp.fft.ifftshift(imgs, axes=(-2, -1)))
    f2d = jnp.fft.fftshift(f2d, axes=-2)
    ky = (jnp.arange(D_) - half).astype(jnp.float32)
    kx = jnp.arange(KX_).astype(jnp.float32)
    ph = -2.0 * jnp.pi * (hwShiftAngs[:, 0, None, None] * ky[None, :, None]
                          + hwShiftAngs[:, 1, None, None] * kx[None, None, :]) / D_
    f2d = f2d * jnp.exp(1j * ph)
    f2d = f2d * ctf
    ctf2 = ctf * ctf
    gky, gkx = jnp.meshgrid(ky, kx, indexing='ij')
    coords = jnp.stack([gkx, gky, jnp.zeros_like(gkx)], axis=-1)
    pos = jnp.einsum('yxc,bcd->byxd', coords, rotMats)
    x3, y3, z3 = pos[..., 0], pos[..., 1], pos[..., 2]
    neg = x3 < 0
    x3 = jnp.where(neg, -x3, x3)
    y3 = jnp.where(neg, -y3, y3)
    z3 = jnp.where(neg, -z3, z3)
    vals = jnp.where(neg, jnp.conj(f2d), f2d)
    ix = x3
    iy = y3 + half
    iz = z3 + half
    x0 = jnp.floor(ix); y0 = jnp.floor(iy); z0 = jnp.floor(iz)
    fx = ix - x0; fy = iy - y0; fz = iz - z0
    acc = jnp.zeros((4, D_, D_, KX_), dtype=jnp.float32)
    vr = jnp.real(vals).reshape(-1)
    vi = jnp.imag(vals).reshape(-1)
    c2 = jnp.broadcast_to(ctf2, vals.shape).reshape(-1)
    ones = jnp.ones_like(vr)
    for dz in (0, 1):
        zz = (z0 + dz).astype(jnp.int32)
        wz = (1.0 - fz) if dz == 0 else fz
        for dy in (0, 1):
            yy = (y0 + dy).astype(jnp.int32)
            wy = (1.0 - fy) if dy == 0 else fy
            for dx in (0, 1):
                xx = (x0 + dx).astype(jnp.int32)
                wx = (1.0 - fx) if dx == 0 else fx
                w = wz * wy * wx
                valid = (xx >= 0) & (xx < KX_) & (yy >= 0) & (yy < D_) & (zz >= 0) & (zz < D_)
                w = jnp.where(valid, w, 0.0).reshape(-1)
                xc = jnp.clip(xx, 0, KX_ - 1).reshape(-1)
                yc = jnp.clip(yy, 0, D_ - 1).reshape(-1)
                zc = jnp.clip(zz, 0, D_ - 1).reshape(-1)
                upd = jnp.stack([w * vr, w * vi, w * ones, w * c2], axis=0)
                comp = jnp.broadcast_to(jnp.arange(4, dtype=jnp.int32)[:, None], upd.shape)
                acc = acc.at[comp.reshape(-1),
                             jnp.tile(zc, 4), jnp.tile(yc, 4), jnp.tile(xc, 4)].add(upd.reshape(-1))
    return acc


def _add_body(a_ref, b_ref, o_ref):
    o_ref[...] = a_ref[...] + b_ref[...]


def kernel(imgs, ctf, rotMats, hwShiftAngs, numerator, weights, ctfsq):
    vol = _accum_vol(imgs, ctf, rotMats, hwShiftAngs)
    base = jnp.concatenate([numerator, weights[None], ctfsq[None]], axis=0)
    n = 4 * _D * _D * _KX
    rows = n // 128
    blk = 1024
    a = vol.reshape(rows, 128)
    b = base.reshape(rows, 128)
    out = pl.pallas_call(
        _add_body,
        grid=(rows // blk,),
        in_specs=[pl.BlockSpec((blk, 128), lambda i: (i, 0)),
                  pl.BlockSpec((blk, 128), lambda i: (i, 0))],
        out_specs=pl.BlockSpec((blk, 128), lambda i: (i, 0)),
        out_shape=jax.ShapeDtypeStruct((rows, 128), jnp.float32),
    )(a, b)
    return out.reshape(4, _D, _D, _KX)


# speedup vs baseline: 7.8201x; 7.8201x over previous
import functools

import jax
import jax.numpy as jnp
from jax import lax
from jax.experimental import pallas as pl
from jax.experimental.pallas import tpu as pltpu
from jax.experimental.pallas import tpu_sc as plsc

_D = 256
_KX = _D // 2 + 1
_B = 64
_HALF = _D // 2
_NROWS = _B * _KX
_NSUB = 16
_ROWS_PER_SUB = _NROWS // _NSUB
_ZS = 13
_NSLAB = 20
_NPASS = _NSLAB // 2
_PLANE = _D * _KX
_NCELL = _ZS * _PLANE
_SENT = 64
_NGRP = 32
_ZCHUNK = _NCELL // _NSUB


def _make_kernel():
    mesh = plsc.VectorSubcoreMesh(core_axis_name="c", subcore_axis_name="s")

    @functools.partial(
        pl.kernel,
        out_type=jax.ShapeDtypeStruct((4, _NSLAB * _NCELL), jnp.float32),
        mesh=mesh,
        compiler_params=pltpu.CompilerParams(use_tc_tiling_on_sc=False),
        scratch_types=[
            pltpu.VMEM((8, _D), jnp.float32),
            pltpu.VMEM((_NGRP, 128), jnp.int32),
            pltpu.VMEM((_NGRP * 128,), jnp.float32),
            pltpu.VMEM((_NGRP * 128,), jnp.float32),
            pltpu.VMEM((_NGRP * 128,), jnp.float32),
            pltpu.VMEM((_NGRP * 128,), jnp.float32),
            pltpu.SMEM((1,), jnp.int32),
            pltpu.VMEM_SHARED((_NCELL + _SENT,), jnp.float32),
            pltpu.VMEM_SHARED((_NCELL + _SENT,), jnp.float32),
            pltpu.VMEM_SHARED((_NCELL + _SENT,), jnp.float32),
            pltpu.VMEM_SHARED((_NCELL + _SENT,), jnp.float32),
        ],
    )
    def bp(pts_hbm, zero_hbm, out_hbm,
           row_v, idx_v, bvr, bvi, bw, bc2, voff_ref,
           acc_r, acc_i, acc_w, acc_c):
        c = lax.axis_index("c")
        s = lax.axis_index("s")
        iota_i = lax.iota(jnp.int32, 16)
        sents = [jnp.int32(_NCELL + 16 * k) + iota_i for k in range(4)]
        accs = (acc_r, acc_i, acc_w, acc_c)

        def _refill():
            for j in range(_NGRP):
                for cg in range(8):
                    idx_v[j, pl.ds(cg * 16, 16)] = sents[cg & 3]

        def _drain():
            for j in range(_NGRP):
                sl = pl.ds(j * 128, 128)
                pltpu.sync_copy(bvr.at[sl], acc_r.at[idx_v.at[j]], add=True)
                pltpu.sync_copy(bvi.at[sl], acc_i.at[idx_v.at[j]], add=True)
                pltpu.sync_copy(bw.at[sl], acc_w.at[idx_v.at[j]], add=True)
                pltpu.sync_copy(bc2.at[sl], acc_c.at[idx_v.at[j]], add=True)
            _refill()
            voff_ref[0] = 0

        _refill()

        def pass_body(p, carry):
            slab = 2 * p + c
            zlo_i = slab * _ZS
            zhi_i = zlo_i + _ZS
            zlo_f = zlo_i.astype(jnp.float32)
            wlo = zlo_f - 129.0
            whi = zlo_f + (_ZS - 128.0)

            for k in range(4):
                pltpu.sync_copy(zero_hbm, accs[k].at[pl.ds(s * _ZCHUNK,
                                                           _ZCHUNK)])
            plsc.subcore_barrier()
            voff_ref[0] = 0

            def row_body(r, carry2):
                @pl.when(voff_ref[0] > _NGRP - 16)
                def _():
                    _drain()

                pltpu.sync_copy(pts_hbm.at[s * _ROWS_PER_SUB + r], row_v)

                for v in range(16):
                    sl = pl.ds(v * 16, 16)
                    glo = row_v[6, sl][0]
                    ghi = row_v[7, sl][0]
                    hitp = (ghi >= wlo) & (glo < whi)
                    hitm = (-glo >= wlo) & (-ghi < whi)

                    @pl.when(hitp | hitm)
                    def _(v=v, sl=sl):
                        x3 = row_v[3, sl]
                        y3 = row_v[4, sl]
                        z3 = row_v[5, sl]
                        sgn = jnp.where(x3 < 0.0, jnp.float32(-1.0),
                                        jnp.float32(1.0))
                        x3f = x3 * sgn
                        iyf = y3 * sgn + 128.0
                        izf = z3 * sgn + 128.0
                        x0 = x3f.astype(jnp.int32)
                        y0 = (iyf + 512.0).astype(jnp.int32) - 512
                        z0 = (izf + 512.0).astype(jnp.int32) - 512
                        fx = x3f - x0.astype(jnp.float32)
                        fy = iyf - y0.astype(jnp.float32)
                        fz = izf - z0.astype(jnp.float32)
                        wx = (1.0 - fx, fx)
                        wy = (1.0 - fy, fy)
                        wz = (1.0 - fz, fz)
                        vr = row_v[0, sl]
                        vi = row_v[1, sl] * sgn
                        c2 = row_v[2, sl]
                        g = voff_ref[0]
                        gbase = g * 128
                        cc = 0
                        for dz in (0, 1):
                            zz = z0 + dz
                            mz = (zz >= zlo_i) & (zz < zhi_i)
                            zoff = (zz - zlo_i) * _PLANE
                            for dy in (0, 1):
                                yy = y0 + dy
                                my = mz & (yy >= 0) & (yy < _D)
                                yoff = zoff + yy * _KX
                                for dx in (0, 1):
                                    xx = x0 + dx
                                    ok = my & (xx < _KX)
                                    w = wz[dz] * wy[dy] * wx[dx]
                                    cell = yoff + xx
                                    cellm = jnp.where(ok, cell, sents[cc & 3])
                                    idx_v[g, pl.ds(cc * 16, 16)] = cellm
                                    bsl = pl.ds(gbase + cc * 16, 16)
                                    bvr[bsl] = w * vr
                                    bvi[bsl] = w * vi
                                    bw[bsl] = w
                                    bc2[bsl] = w * c2
                                    cc += 1
                        voff_ref[0] = g + 1
                return carry2

            lax.fori_loop(0, _ROWS_PER_SUB, row_body, 0)
            _drain()
            plsc.subcore_barrier()

            base = slab * _NCELL + s * _ZCHUNK
            for k in range(4):
                pltpu.sync_copy(accs[k].at[pl.ds(s * _ZCHUNK, _ZCHUNK)],
                                out_hbm.at[k, pl.ds(base, _ZCHUNK)])
            plsc.subcore_barrier()
            return carry

        lax.fori_loop(0, _NPASS, pass_body, 0)

    return bp


_BP = _make_kernel()


def kernel(imgs, ctf, rotMats, hwShiftAngs, numerator, weights, ctfsq):
    f2d = jnp.fft.rfft2(jnp.fft.ifftshift(imgs, axes=(-2, -1)))
    f2d = jnp.fft.fftshift(f2d, axes=-2)
    ky = (jnp.arange(_D) - _HALF).astype(jnp.float32)
    kx = jnp.arange(_KX).astype(jnp.float32)
    ph = -2.0 * jnp.pi * (hwShiftAngs[:, 0, None, None] * ky[None, :, None]
                          + hwShiftAngs[:, 1, None, None] * kx[None, None, :]) / _D
    f2d = f2d * jnp.exp(1j * ph)
    f2d = f2d * ctf
    c2 = ctf * ctf

    gky, gkx = jnp.meshgrid(ky, kx, indexing='ij')
    coords = jnp.stack([gkx, gky, jnp.zeros_like(gkx)], axis=-1)
    pos = jnp.einsum('yxc,bcd->byxd', coords, rotMats)
    x3, y3, z3 = pos[..., 0], pos[..., 1], pos[..., 2]

    def t(a):
        return a.transpose(0, 2, 1)

    z3t = t(z3)
    z3g = z3t.reshape(_B, _KX, 16, 16)
    glo = jnp.repeat(z3g.min(axis=-1), 16, axis=-1)
    ghi = jnp.repeat(z3g.max(axis=-1), 16, axis=-1)
    pts = jnp.stack([t(jnp.real(f2d)), t(jnp.imag(f2d)), t(c2),
                     t(x3), t(y3), z3t, glo, ghi], axis=2)
    pts = pts.reshape(_NROWS, 8, _D).astype(jnp.float32)

    zero = jnp.zeros((_ZCHUNK,), jnp.float32)

    flat = _BP(pts, zero)
    vol = flat.reshape(4, _NSLAB * _ZS, _D, _KX)[:, :_D]
    base = jnp.concatenate([numerator, weights[None], ctfsq[None]], axis=0)
    return base + vol
